# baseline (speedup 1.0000x reference)
import functools

import jax
import jax.numpy as jnp
from jax import lax
from jax.experimental import pallas as pl
from jax.experimental.pallas import tpu as pltpu
from jax.experimental.pallas import tpu_sc as plsc

B = 16384
D = 16
NG = 1000000
NJ = 1000000
NC = 2
NS = 16
NW = NC * NS
BPW = B // NW
NCH = 4
CH = BPW // NCH


def _mf_body(gid_hbm, jid_hbm, zg_hbm, zj_hbm,
             gb1_hbm, jb1_hbm, gb2_hbm, jb2_hbm, w_hbm,
             out_hbm,
             gidx, jidx, gsh, jsh, gbuf, jbuf,
             gb1v, jb1v, gb2v, jb2v, wv, outv, sem, semb):
    wid = lax.axis_index("s") * NC + lax.axis_index("c")
    base = wid * BPW

    descs = [
        pltpu.async_copy(w_hbm, wv, sem),
        pltpu.async_copy(gid_hbm.at[pl.ds(base, BPW)], gidx, sem),
        pltpu.async_copy(jid_hbm.at[pl.ds(base, BPW)], jidx, sem),
    ]
    for d_ in descs:
        d_.wait()

    bdescs = []
    for c in range(NCH):
        sl = pl.ds(c * CH, CH)
        gi = gidx.at[sl]
        ji = jidx.at[sl]
        bdescs.append(pltpu.async_copy(gb1_hbm.at[gi], gb1v.at[sl], semb))
        bdescs.append(pltpu.async_copy(jb1_hbm.at[ji], jb1v.at[sl], semb))
        bdescs.append(pltpu.async_copy(gb2_hbm.at[gi], gb2v.at[sl], semb))
        bdescs.append(pltpu.async_copy(jb2_hbm.at[ji], jb2v.at[sl], semb))

    def sh_body(d, carry):
        off = d * NG
        for c in range(NCH):
            for k in range(CH // 16):
                sl = pl.ds(c * CH + k * 16, 16)
                dst = pl.ds(k * 16, 16)
                gsh[d, c, dst] = gidx[sl] + off
                jsh[d, c, dst] = jidx[sl] + off
        return carry

    lax.fori_loop(0, 2 * D, sh_body, 0)

    descs = []
    for c in range(NCH):
        for d in range(2 * D):
            dst = pl.ds(c * CH, CH)
            descs.append(pltpu.async_copy(
                zg_hbm.at[gsh.at[d, c]], gbuf.at[d, dst], sem))
            descs.append(pltpu.async_copy(
                zj_hbm.at[jsh.at[d, c]], jbuf.at[d, dst], sem))
    for d_ in bdescs:
        d_.wait()
    for d_ in descs:
        d_.wait()

    cv = wv[0]

    def blk_body(blk, carry):
        bsl = pl.ds(blk * 16, 16)
        acc = cv + gb1v[bsl] + gb2v[bsl] + jb1v[bsl] + jb2v[bsl]
        for d in range(D):
            a = gbuf[d, bsl]
            b = jbuf[d, bsl]
            c2 = gbuf[D + d, bsl]
            e = jbuf[D + d, bsl]
            acc = acc + a * b + c2 * e
        outv[bsl] = acc
        return carry

    lax.fori_loop(0, BPW // 16, blk_body, 0)

    pltpu.sync_copy(outv, out_hbm.at[pl.ds(base, BPW)])


_mf_call = functools.partial(
    pl.kernel,
    out_type=jax.ShapeDtypeStruct((B,), jnp.float32),
    mesh=plsc.VectorSubcoreMesh(core_axis_name="c", subcore_axis_name="s",
                                num_cores=NC, num_subcores=NS),
    scratch_types=[
        pltpu.VMEM((BPW,), jnp.int32),
        pltpu.VMEM((BPW,), jnp.int32),
        pltpu.VMEM((2 * D, NCH, CH), jnp.int32),
        pltpu.VMEM((2 * D, NCH, CH), jnp.int32),
        pltpu.VMEM((2 * D, BPW), jnp.float32),
        pltpu.VMEM((2 * D, BPW), jnp.float32),
        pltpu.VMEM((BPW,), jnp.float32),
        pltpu.VMEM((BPW,), jnp.float32),
        pltpu.VMEM((BPW,), jnp.float32),
        pltpu.VMEM((BPW,), jnp.float32),
        pltpu.VMEM((1, D), jnp.float32),
        pltpu.VMEM((BPW,), jnp.float32),
        pltpu.SemaphoreType.DMA,
        pltpu.SemaphoreType.DMA,
    ],
    compiler_params=pltpu.CompilerParams(needs_layout_passes=False,
                                         use_tc_tiling_on_sc=False),
)(_mf_body)


def kernel(geek_id, job_id, geek_emb1, job_emb1, geek_emb2, job_emb2,
           geek_b1, job_b1, geek_b2, job_b2, W1, b1, W2, b2, miu1, miu2):
    w = (W1 @ W2)[:, 0]
    const = (b1 @ W2)[0] + b2[0] + miu1 + miu2
    wpack = jnp.full((1, D), const, jnp.float32)
    rt = jnp.sqrt(jnp.abs(w))
    sg = jnp.sign(w) * rt
    zg = jnp.concatenate([(geek_emb1 * sg[:D]).T.reshape(-1),
                          (geek_emb2 * sg[D:]).T.reshape(-1)])
    zj = jnp.concatenate([(job_emb1 * rt[:D]).T.reshape(-1),
                          (job_emb2 * rt[D:]).T.reshape(-1)])
    return _mf_call(geek_id.astype(jnp.int32), job_id.astype(jnp.int32),
                    zg, zj,
                    geek_b1[:, 0], job_b1[:, 0], geek_b2[:, 0], job_b2[:, 0],
                    wpack)


# speedup vs baseline: 3.6520x; 3.6520x over previous
import functools

import jax
import jax.numpy as jnp
from jax import lax
from jax.experimental import pallas as pl
from jax.experimental.pallas import tpu as pltpu
from jax.experimental.pallas import tpu_sc as plsc

B = 16384
D = 16
NG = 1000000
NJ = 1000000
NC = 2
NS = 16
NW = NC * NS
BPW = B // NW
NCH = 4
CH = BPW // NCH


def _mf_body(gid_hbm, jid_hbm, g1_hbm, j1_hbm, g2_hbm, j2_hbm,
             gb1_hbm, jb1_hbm, gb2_hbm, jb2_hbm, w_hbm,
             out_hbm,
             gidx, jidx, g1v, j1v, g2v, j2v,
             gb1v, jb1v, gb2v, jb2v, wv, outv, sem):
    wid = lax.axis_index("s") * NC + lax.axis_index("c")
    base = wid * BPW

    descs = [pltpu.async_copy(w_hbm, wv, sem)]
    for c in range(NCH):
        descs.append(pltpu.async_copy(
            gid_hbm.at[pl.ds(base + c * CH, CH)], gidx.at[c], sem))
        descs.append(pltpu.async_copy(
            jid_hbm.at[pl.ds(base + c * CH, CH)], jidx.at[c], sem))
    for d_ in descs:
        d_.wait()

    descs = []
    for c in range(NCH):
        gi = gidx.at[c]
        ji = jidx.at[c]
        sl = pl.ds(c * CH, CH)
        descs.append(pltpu.async_copy(g1_hbm.at[gi], g1v.at[sl], sem))
        descs.append(pltpu.async_copy(j1_hbm.at[ji], j1v.at[sl], sem))
        descs.append(pltpu.async_copy(g2_hbm.at[gi], g2v.at[sl], sem))
        descs.append(pltpu.async_copy(j2_hbm.at[ji], j2v.at[sl], sem))
        descs.append(pltpu.async_copy(gb1_hbm.at[gi], gb1v.at[sl], sem))
        descs.append(pltpu.async_copy(jb1_hbm.at[ji], jb1v.at[sl], sem))
        descs.append(pltpu.async_copy(gb2_hbm.at[gi], gb2v.at[sl], sem))
        descs.append(pltpu.async_copy(jb2_hbm.at[ji], jb2v.at[sl], sem))
    for d_ in descs:
        d_.wait()

    iota16 = lax.iota(jnp.int32, 16)
    cv = wv[2]
    wa_vec = wv[0]
    wb_vec = wv[1]
    was = [wa_vec[d] for d in range(D)]
    wbs = [wb_vec[d] for d in range(D)]

    def blk_body(blk, carry):
        pv = iota16 + blk * 16
        bsl = pl.ds(blk * 16, 16)
        acc = cv + gb1v[bsl] + gb2v[bsl] + jb1v[bsl] + jb2v[bsl]
        for d in range(D):
            dsplat = jnp.full((16,), d, jnp.int32)
            a = plsc.load_gather(g1v, [pv, dsplat])
            b = plsc.load_gather(j1v, [pv, dsplat])
            c2 = plsc.load_gather(g2v, [pv, dsplat])
            e = plsc.load_gather(j2v, [pv, dsplat])
            acc = acc + a * b * was[d] + c2 * e * wbs[d]
        outv[bsl] = acc
        return carry

    lax.fori_loop(0, BPW // 16, blk_body, 0)

    pltpu.sync_copy(outv, out_hbm.at[pl.ds(base, BPW)])


_mf_call = functools.partial(
    pl.kernel,
    out_type=jax.ShapeDtypeStruct((B,), jnp.float32),
    mesh=plsc.VectorSubcoreMesh(core_axis_name="c", subcore_axis_name="s",
                                num_cores=NC, num_subcores=NS),
    scratch_types=[
        pltpu.VMEM((NCH, CH), jnp.int32),
        pltpu.VMEM((NCH, CH), jnp.int32),
        pltpu.VMEM((BPW, D), jnp.float32),
        pltpu.VMEM((BPW, D), jnp.float32),
        pltpu.VMEM((BPW, D), jnp.float32),
        pltpu.VMEM((BPW, D), jnp.float32),
        pltpu.VMEM((BPW,), jnp.float32),
        pltpu.VMEM((BPW,), jnp.float32),
        pltpu.VMEM((BPW,), jnp.float32),
        pltpu.VMEM((BPW,), jnp.float32),
        pltpu.VMEM((3, D), jnp.float32),
        pltpu.VMEM((BPW,), jnp.float32),
        pltpu.SemaphoreType.DMA,
    ],
    compiler_params=pltpu.CompilerParams(needs_layout_passes=False,
                                         use_tc_tiling_on_sc=False),
)(_mf_body)


def kernel(geek_id, job_id, geek_emb1, job_emb1, geek_emb2, job_emb2,
           geek_b1, job_b1, geek_b2, job_b2, W1, b1, W2, b2, miu1, miu2):
    w = (W1 @ W2)[:, 0]
    const = (b1 @ W2)[0] + b2[0] + miu1 + miu2
    wpack = jnp.stack([w[:D], w[D:], jnp.full((D,), const, jnp.float32)])
    return _mf_call(geek_id.astype(jnp.int32), job_id.astype(jnp.int32),
                    geek_emb1, job_emb1, geek_emb2, job_emb2,
                    geek_b1[:, 0], job_b1[:, 0], geek_b2[:, 0], job_b2[:, 0],
                    wpack)
